# baseline (speedup 1.0000x reference)
import functools

import numpy as np
import jax
import jax.numpy as jnp
from jax import lax
from jax.experimental import pallas as pl
from jax.experimental.pallas import tpu as pltpu
from jax.experimental.pallas import tpu_sc as plsc

B = 64
N = 20
NUM_SPECIES = 4
NPB = NUM_SPECIES * (NUM_SPECIES + 1) // 2
RCR = 5.2
RCA = 3.5
ETA_R = 16.0
ZETA = 32.0
ETA_A = 8.0
SHF_R = [0.9, 1.16875, 1.4375, 1.70625, 1.975, 2.24375, 2.5125, 2.78125,
         3.05, 3.31875, 3.5875, 3.85625, 4.125, 4.39375, 4.6625, 4.93125]
SHF_Z = [0.19634954, 0.58904862, 0.9817477, 1.3744468, 1.7671459,
         2.1598449, 2.552544, 2.9452431]
SHF_A = [0.9, 1.55, 2.2, 2.85]
COS_Z = [float(np.cos(z)) for z in SHF_Z]
SIN_Z = [float(np.sin(z)) for z in SHF_Z]

NRAD = 16
NSUB = 32
RFEAT = NUM_SPECIES * NRAD

NP2 = N * N
NCHUNK = NP2 // 16
NTRI = N * (N - 1) // 2
NTRI_PAD = 192
NTCHUNK = NTRI_PAD // 16
NROW = N * NSUB

NC, NS = 2, 16
NW = NC * NS
MOLS_PER_W = B // NW

TC_BLK = 16

_ss = np.linspace(1e-6, 0.5, 4001)
_P3, _P2, _P1, _P0 = [float(c) for c in
                      np.polyfit(_ss * _ss, -np.sin(np.pi * _ss) / _ss, 3)]

_pj, _pk = np.triu_indices(N, 1)
PAIR_J = np.concatenate([_pj, np.zeros(NTRI_PAD - NTRI, np.int64)]).astype(np.int32)
PAIR_K = np.concatenate([_pk, np.zeros(NTRI_PAD - NTRI, np.int64)]).astype(np.int32)


def _sel_consts():
    nc = NPB * NSUB
    bc = np.zeros((NPB, nc), np.float32)
    msk = np.zeros((NROW, nc), np.float32)
    sel = np.zeros((N, NROW), np.float32)
    b2 = np.zeros((2 * NPB, nc), np.float32)
    msk2 = np.zeros((NROW // 2, nc), np.float32)
    sel2 = np.zeros((N, NROW // 2), np.float32)
    for pb in range(NPB):
        for s in range(NSUB):
            c = pb * NSUB + s
            bc[pb, c] = 1.0
            b2[(s % 2) * NPB + pb, c] = 1.0
    for r in range(NROW):
        for pb in range(NPB):
            msk[r, pb * NSUB + (r % NSUB)] = 1.0
        sel[r // NSUB, r] = 1.0
    for t in range(NROW // 2):
        for pb in range(NPB):
            for par in range(2):
                s = (t % 16) * 2 + par
                msk2[t, pb * NSUB + s] = 1.0
        sel2[t // 16, t] = 1.0
    return bc, msk, sel, b2, msk2, sel2


SEL_BC, SEL_MSK, SEL_SEL, SEL_B2, SEL_MSK2, SEL_SEL2 = _sel_consts()


def _rsqrt_nr(x, iters=3):
    i = plsc.bitcast(x, jnp.int32)
    i = 0x5F3759DF - (i >> 1)
    y = plsc.bitcast(i, jnp.float32)
    for _ in range(iters):
        y = y * (1.5 - 0.5 * x * y * y)
    return y


def _cutoff(d, rc):
    s = d * (1.0 / rc) - 0.5
    t = s * s
    f = s * (((_P3 * t + _P2) * t + _P1) * t + _P0)
    return jnp.where(d <= rc, 0.5 * f + 0.5, 0.0)


def _sc_body(labels_hbm, coords_hbm, pj_hbm, pk_hbm,
             rad_hbm, stg1_hbm, stg2_hbm, pb_hbm,
             lab_v, xyz_v, pj_v, pk_v, dst_v, inv_v, fca_v, pb_v, rad_v,
             stg1_v, stg2_v):
    wid = lax.axis_index("s") * NC + lax.axis_index("c")
    pltpu.sync_copy(pj_hbm, pj_v)
    pltpu.sync_copy(pk_hbm, pk_v)
    iota = lax.iota(jnp.int32, 16)
    zeros16 = jnp.zeros((16,), jnp.float32)

    for m in range(MOLS_PER_W):
        mol = wid * MOLS_PER_W + m
        pltpu.sync_copy(labels_hbm.at[mol], lab_v)
        pltpu.sync_copy(coords_hbm.at[mol], xyz_v)

        @plsc.parallel_loop(0, N * RFEAT // 16, unroll=4)
        def zero_body(c):
            rad_v[pl.ds(c * 16, 16)] = zeros16

        @plsc.parallel_loop(0, NCHUNK, unroll=2)
        def tab_body(c):
            jk = c * 16 + iota
            iv = jk // N
            jv = jk - iv * N
            dims0 = jnp.zeros((16,), jnp.int32)
            xi = plsc.load_gather(xyz_v, [iv, dims0])
            xj = plsc.load_gather(xyz_v, [jv, dims0])
            yi = plsc.load_gather(xyz_v, [iv, dims0 + 1])
            yj = plsc.load_gather(xyz_v, [jv, dims0 + 1])
            zi = plsc.load_gather(xyz_v, [iv, dims0 + 2])
            zj = plsc.load_gather(xyz_v, [jv, dims0 + 2])
            dx = xi - xj
            dy = yi - yj
            dz = zi - zj
            d2 = dx * dx + dy * dy + dz * dz + 1e-12
            rinv = _rsqrt_nr(d2)
            d = d2 * rinv
            diag = iv == jv
            d = jnp.where(diag, 1e6, d)
            rinv = jnp.where(diag, 1e-6, rinv)
            sl = pl.ds(c * 16, 16)
            dst_v[sl] = d
            inv_v[sl] = rinv
            fca_v[sl] = _cutoff(d, RCA)

        @plsc.parallel_loop(0, NCHUNK, unroll=2)
        def rad_body(c):
            jk = c * 16 + iota
            iv = jk // N
            jv = jk - iv * N
            sl = pl.ds(c * 16, 16)
            d = dst_v[sl]
            w = 0.25 * _cutoff(d, RCR)
            sj = plsc.load_gather(lab_v, [jv])
            base = iv * RFEAT + sj * NRAD
            idxr = [base + r for r in range(8)]
            for r in range(NRAD):
                t = d - SHF_R[r]
                val = w * jnp.exp((-ETA_R) * (t * t))
                plsc.addupdate_scatter(
                    rad_v.at[pl.ds((r // 8) * 8, N * RFEAT - 8)],
                    [idxr[r % 8]], val)

        def make_ang(c, store):
            sl = pl.ds(c * 16, 16)
            pj = pj_v[sl]
            pk = pk_v[sl]
            sj = plsc.load_gather(lab_v, [pj])
            sk = plsc.load_gather(lab_v, [pk])
            smin = jnp.minimum(sj, sk)
            smax = jnp.maximum(sj, sk)
            pb_v[sl] = ((smin * (2 * NUM_SPECIES - 1 - smin)) >> 1) + smax
            djk = plsc.load_gather(dst_v, [pj * N + pk])
            djk2 = djk * djk
            valid = pj < pk

            @plsc.parallel_loop(0, N, unroll=2)
            def ang_inner(i):
                idxj = i * N + pj
                idxk = i * N + pk
                d_ij = plsc.load_gather(dst_v, [idxj])
                d_ik = plsc.load_gather(dst_v, [idxk])
                i_ij = plsc.load_gather(inv_v, [idxj])
                i_ik = plsc.load_gather(inv_v, [idxk])
                f_ij = plsc.load_gather(fca_v, [idxj])
                f_ik = plsc.load_gather(fca_v, [idxk])
                dot = 0.5 * (d_ij * d_ij + d_ik * d_ik - djk2)
                ct = 0.95 * dot * i_ij * i_ik
                ct = jnp.minimum(jnp.maximum(ct, -0.95), 0.95)
                st2 = 1.0 - ct * ct
                st = st2 * _rsqrt_nr(st2, iters=2)
                rmean = 0.5 * (d_ij + d_ik)
                w2 = jnp.where(valid, 2.0 * f_ij * f_ik, 0.0)
                for a in range(4):
                    ta = rmean - SHF_A[a]
                    f2w = w2 * jnp.exp((-ETA_A) * (ta * ta))
                    for z in range(8):
                        u = 0.5 + (0.5 * COS_Z[z]) * ct + (0.5 * SIN_Z[z]) * st
                        u = u * u
                        u = u * u
                        u = u * u
                        u = u * u
                        u = u * u
                        store(i, c, a * 8 + z, f2w * u)

        def store_lo(i, c, s, val):
            stg1_v[i * NSUB + s, pl.ds(c * 16, 16)] = val

        def store_hi(i, c, s, val):
            stg2_v[i * (NSUB // 2) + s // 2,
                   pl.ds((s % 2) * 64 + (c - 8) * 16, 16)] = val

        def ang_lo(c, carry):
            make_ang(c, store_lo)
            return carry
        lax.fori_loop(0, 8, ang_lo, 0)

        def ang_hi(c, carry):
            make_ang(c, store_hi)
            return carry
        lax.fori_loop(8, NTCHUNK, ang_hi, 0)

        pltpu.sync_copy(rad_v, rad_hbm.at[mol])
        pltpu.sync_copy(stg1_v, stg1_hbm.at[mol])
        pltpu.sync_copy(stg2_v, stg2_hbm.at[mol])
        pltpu.sync_copy(pb_v, pb_hbm.at[mol])


def _tc_seg_body(oh1_ref, oh2_ref, stg1_ref, stg2_ref,
                 bc_ref, msk_ref, sel_ref, b2_ref, msk2_ref, sel2_ref,
                 out_ref):
    bc = bc_ref[...]
    msk = msk_ref[...]
    sel = sel_ref[...]
    b2 = b2_ref[...]
    msk2 = msk2_ref[...]
    sel2 = sel2_ref[...]
    dot = lambda a, b: jnp.dot(a, b, preferred_element_type=jnp.float32)
    for m in range(TC_BLK):
        p1 = dot(stg1_ref[m], oh1_ref[m])
        ang1 = dot(sel, dot(p1, bc) * msk)
        p2 = dot(stg2_ref[m], oh2_ref[m])
        ang2 = dot(sel2, dot(p2, b2) * msk2)
        out_ref[m] = ang1 + ang2


@jax.jit
def _aev(labels, coords, pj, pk):
    mesh = plsc.VectorSubcoreMesh(core_axis_name="c", subcore_axis_name="s")
    sc_fn = pl.kernel(
        _sc_body,
        out_type=(
            jax.ShapeDtypeStruct((B, N * RFEAT), jnp.float32),
            jax.ShapeDtypeStruct((B, NROW, 128), jnp.float32),
            jax.ShapeDtypeStruct((B, NROW // 2, 128), jnp.float32),
            jax.ShapeDtypeStruct((B, NTRI_PAD), jnp.int32),
        ),
        mesh=mesh,
        scratch_types=[
            pltpu.VMEM((N,), jnp.int32),
            pltpu.VMEM((N, 3), jnp.float32),
            pltpu.VMEM((NTRI_PAD,), jnp.int32),
            pltpu.VMEM((NTRI_PAD,), jnp.int32),
            pltpu.VMEM((NP2,), jnp.float32),
            pltpu.VMEM((NP2,), jnp.float32),
            pltpu.VMEM((NP2,), jnp.float32),
            pltpu.VMEM((NTRI_PAD,), jnp.int32),
            pltpu.VMEM((N * RFEAT,), jnp.float32),
            pltpu.VMEM((NROW, 128), jnp.float32),
            pltpu.VMEM((NROW // 2, 128), jnp.float32),
        ],
        compiler_params=pltpu.CompilerParams(needs_layout_passes=False),
    )
    rad, stg1, stg2, pb = sc_fn(labels, coords, pj, pk)

    oh = (pb[:, :, None] == jnp.arange(NPB, dtype=jnp.int32)
          ).astype(jnp.float32)
    oh1 = oh[:, :128, :]
    hi = oh[:, 128:, :]
    zz = jnp.zeros((B, 64, NPB), jnp.float32)
    oh2 = jnp.concatenate([jnp.concatenate([hi, zz], 2),
                           jnp.concatenate([zz, hi], 2)], 1)
    full = lambda g: (0, 0)
    ang = pl.pallas_call(
        _tc_seg_body,
        grid=(B // TC_BLK,),
        in_specs=[
            pl.BlockSpec((TC_BLK, 128, NPB), lambda g: (g, 0, 0)),
            pl.BlockSpec((TC_BLK, 128, 2 * NPB), lambda g: (g, 0, 0)),
            pl.BlockSpec((TC_BLK, NROW, 128), lambda g: (g, 0, 0)),
            pl.BlockSpec((TC_BLK, NROW // 2, 128), lambda g: (g, 0, 0)),
            pl.BlockSpec((NPB, NPB * NSUB), full),
            pl.BlockSpec((NROW, NPB * NSUB), full),
            pl.BlockSpec((N, NROW), full),
            pl.BlockSpec((2 * NPB, NPB * NSUB), full),
            pl.BlockSpec((NROW // 2, NPB * NSUB), full),
            pl.BlockSpec((N, NROW // 2), full),
        ],
        out_specs=pl.BlockSpec((TC_BLK, N, NPB * NSUB), lambda g: (g, 0, 0)),
        out_shape=jax.ShapeDtypeStruct((B, N, NPB * NSUB), jnp.float32),
    )(oh1, oh2, stg1, stg2,
      jnp.asarray(SEL_BC), jnp.asarray(SEL_MSK), jnp.asarray(SEL_SEL),
      jnp.asarray(SEL_B2), jnp.asarray(SEL_MSK2), jnp.asarray(SEL_SEL2))
    return jnp.concatenate([rad.reshape(B, N, RFEAT), ang], axis=-1)


def kernel(labels_tensor, coords_tensor):
    pj = jnp.asarray(PAIR_J)
    pk = jnp.asarray(PAIR_K)
    aev = _aev(labels_tensor, coords_tensor, pj, pk)
    return (labels_tensor, aev)


# speedup vs baseline: 1.0119x; 1.0119x over previous
import numpy as np
import jax
import jax.numpy as jnp
from jax import lax
from jax.experimental import pallas as pl
from jax.experimental.pallas import tpu as pltpu
from jax.experimental.pallas import tpu_sc as plsc

B = 64
N = 20
NUM_SPECIES = 4
NPB = NUM_SPECIES * (NUM_SPECIES + 1) // 2
RCR = 5.2
RCA = 3.5
ETA_R = 16.0
ZETA = 32.0
ETA_A = 8.0
SHF_R = [0.9, 1.16875, 1.4375, 1.70625, 1.975, 2.24375, 2.5125, 2.78125,
         3.05, 3.31875, 3.5875, 3.85625, 4.125, 4.39375, 4.6625, 4.93125]
SHF_Z = [0.19634954, 0.58904862, 0.9817477, 1.3744468, 1.7671459,
         2.1598449, 2.552544, 2.9452431]
SHF_A = [0.9, 1.55, 2.2, 2.85]
COS_Z = [float(np.cos(z)) for z in SHF_Z]
SIN_Z = [float(np.sin(z)) for z in SHF_Z]

NRAD = 16
NSUB = 32
RFEAT = NUM_SPECIES * NRAD

NP2 = N * N
NCHUNK = NP2 // 16
NTRI = N * (N - 1) // 2
NTRI_PAD = 192
NTCHUNK = NTRI_PAD // 16
NROW = N * NSUB

NC, NS = 2, 16
NW = NC * NS
MOLS_PER_W = B // NW

TC_BLK = 8

_ss = np.linspace(1e-6, 0.5, 4001)
_P3, _P2, _P1, _P0 = [float(c) for c in
                      np.polyfit(_ss * _ss, -np.sin(np.pi * _ss) / _ss, 3)]

_pj, _pk = np.triu_indices(N, 1)
PAIR_J = np.concatenate([_pj, np.zeros(NTRI_PAD - NTRI, np.int64)]).astype(np.int32)
PAIR_K = np.concatenate([_pk, np.zeros(NTRI_PAD - NTRI, np.int64)]).astype(np.int32)


def _sel_consts():
    nc = NPB * NSUB
    bc = np.zeros((NPB, nc), np.float32)
    msk = np.zeros((NROW, nc), np.float32)
    sel = np.zeros((N, NROW), np.float32)
    b2 = np.zeros((2 * NPB, nc), np.float32)
    msk2 = np.zeros((NROW // 2, nc), np.float32)
    sel2 = np.zeros((N, NROW // 2), np.float32)
    for pb in range(NPB):
        for s in range(NSUB):
            c = pb * NSUB + s
            bc[pb, c] = 1.0
            b2[(s % 2) * NPB + pb, c] = 1.0
    for r in range(NROW):
        for pb in range(NPB):
            msk[r, pb * NSUB + (r % NSUB)] = 1.0
        sel[r // NSUB, r] = 1.0
    for t in range(NROW // 2):
        for pb in range(NPB):
            for par in range(2):
                s = (t % 16) * 2 + par
                msk2[t, pb * NSUB + s] = 1.0
        sel2[t // 16, t] = 1.0
    return bc, msk, sel, b2, msk2, sel2


SEL_BC, SEL_MSK, SEL_SEL, SEL_B2, SEL_MSK2, SEL_SEL2 = _sel_consts()


def _rsqrt_nr(x, iters=3):
    i = plsc.bitcast(x, jnp.int32)
    i = 0x5F3759DF - (i >> 1)
    y = plsc.bitcast(i, jnp.float32)
    for _ in range(iters):
        y = y * (1.5 - 0.5 * x * y * y)
    return y


def _cutoff(d, rc):
    s = d * (1.0 / rc) - 0.5
    t = s * s
    f = s * (((_P3 * t + _P2) * t + _P1) * t + _P0)
    return jnp.where(d <= rc, 0.5 * f + 0.5, 0.0)


def _sc_body(labels_hbm, coords_hbm, pj_hbm, pk_hbm,
             rad_hbm, stg1_hbm, stg2_hbm, pb_hbm,
             lab_v, xyz_v, pj_v, pk_v, dst_v, inv_v, fca_v, pb_v, rad_v,
             stg1_v, stg2_v):
    wid = lax.axis_index("s") * NC + lax.axis_index("c")
    pltpu.sync_copy(pj_hbm, pj_v)
    pltpu.sync_copy(pk_hbm, pk_v)
    iota = lax.iota(jnp.int32, 16)
    zeros16 = jnp.zeros((16,), jnp.float32)

    for m in range(MOLS_PER_W):
        mol = wid * MOLS_PER_W + m
        pltpu.sync_copy(labels_hbm.at[mol], lab_v)
        pltpu.sync_copy(coords_hbm.at[mol], xyz_v)

        @plsc.parallel_loop(0, N * RFEAT // 16, unroll=4)
        def zero_body(c):
            rad_v[pl.ds(c * 16, 16)] = zeros16

        @plsc.parallel_loop(0, NCHUNK, unroll=2)
        def tab_body(c):
            jk = c * 16 + iota
            iv = jk // N
            jv = jk - iv * N
            dims0 = jnp.zeros((16,), jnp.int32)
            xi = plsc.load_gather(xyz_v, [iv, dims0])
            xj = plsc.load_gather(xyz_v, [jv, dims0])
            yi = plsc.load_gather(xyz_v, [iv, dims0 + 1])
            yj = plsc.load_gather(xyz_v, [jv, dims0 + 1])
            zi = plsc.load_gather(xyz_v, [iv, dims0 + 2])
            zj = plsc.load_gather(xyz_v, [jv, dims0 + 2])
            dx = xi - xj
            dy = yi - yj
            dz = zi - zj
            d2 = dx * dx + dy * dy + dz * dz + 1e-12
            rinv = _rsqrt_nr(d2)
            d = d2 * rinv
            diag = iv == jv
            d = jnp.where(diag, 1e6, d)
            rinv = jnp.where(diag, 1e-6, rinv)
            sl = pl.ds(c * 16, 16)
            dst_v[sl] = d
            inv_v[sl] = rinv
            fca_v[sl] = _cutoff(d, RCA)

        @plsc.parallel_loop(0, NCHUNK, unroll=2)
        def rad_body(c):
            jk = c * 16 + iota
            iv = jk // N
            jv = jk - iv * N
            sl = pl.ds(c * 16, 16)
            d = dst_v[sl]
            w = 0.25 * _cutoff(d, RCR)
            sj = plsc.load_gather(lab_v, [jv])
            base = iv * RFEAT + sj * NRAD
            idxr = [base + r for r in range(8)]
            for r in range(NRAD):
                t = d - SHF_R[r]
                val = w * jnp.exp((-ETA_R) * (t * t))
                plsc.addupdate_scatter(
                    rad_v.at[pl.ds((r // 8) * 8, N * RFEAT - 8)],
                    [idxr[r % 8]], val)

        def make_ang(c, store):
            sl = pl.ds(c * 16, 16)
            pj = pj_v[sl]
            pk = pk_v[sl]
            sj = plsc.load_gather(lab_v, [pj])
            sk = plsc.load_gather(lab_v, [pk])
            smin = jnp.minimum(sj, sk)
            smax = jnp.maximum(sj, sk)
            pb_v[sl] = ((smin * (2 * NUM_SPECIES - 1 - smin)) >> 1) + smax
            djk = plsc.load_gather(dst_v, [pj * N + pk])
            djk2 = djk * djk
            valid = pj < pk

            @plsc.parallel_loop(0, N, unroll=2)
            def ang_inner(i):
                idxj = i * N + pj
                idxk = i * N + pk
                d_ij = plsc.load_gather(dst_v, [idxj])
                d_ik = plsc.load_gather(dst_v, [idxk])
                i_ij = plsc.load_gather(inv_v, [idxj])
                i_ik = plsc.load_gather(inv_v, [idxk])
                f_ij = plsc.load_gather(fca_v, [idxj])
                f_ik = plsc.load_gather(fca_v, [idxk])
                dot = 0.5 * (d_ij * d_ij + d_ik * d_ik - djk2)
                ct = 0.95 * dot * i_ij * i_ik
                ct = jnp.minimum(jnp.maximum(ct, -0.95), 0.95)
                st2 = 1.0 - ct * ct
                st = st2 * _rsqrt_nr(st2, iters=2)
                rmean = 0.5 * (d_ij + d_ik)
                w2 = jnp.where(valid, 2.0 * f_ij * f_ik, 0.0)
                for a in range(4):
                    ta = rmean - SHF_A[a]
                    f2w = w2 * jnp.exp((-ETA_A) * (ta * ta))
                    for z in range(8):
                        u = 0.5 + (0.5 * COS_Z[z]) * ct + (0.5 * SIN_Z[z]) * st
                        u = u * u
                        u = u * u
                        u = u * u
                        u = u * u
                        u = u * u
                        store(i, c, a * 8 + z, f2w * u)

        def store_lo(i, c, s, val):
            stg1_v[i * NSUB + s, pl.ds(c * 16, 16)] = val

        def store_hi(i, c, s, val):
            stg2_v[i * (NSUB // 2) + s // 2,
                   pl.ds((s % 2) * 64 + (c - 8) * 16, 16)] = val

        def ang_lo(c, carry):
            make_ang(c, store_lo)
            return carry
        lax.fori_loop(0, 8, ang_lo, 0)

        def ang_hi(c, carry):
            make_ang(c, store_hi)
            return carry
        lax.fori_loop(8, NTCHUNK, ang_hi, 0)

        pltpu.sync_copy(rad_v, rad_hbm.at[mol])
        pltpu.sync_copy(stg1_v, stg1_hbm.at[mol])
        pltpu.sync_copy(stg2_v, stg2_hbm.at[mol])
        pltpu.sync_copy(pb_v, pb_hbm.at[mol])


def _tc_seg_body(oh1_ref, oh2_ref, stg1_ref, stg2_ref,
                 bc_ref, msk_ref, sel_ref, b2_ref, msk2_ref, sel2_ref,
                 out_ref):
    bc = bc_ref[...]
    msk = msk_ref[...]
    sel = sel_ref[...]
    b2 = b2_ref[...]
    msk2 = msk2_ref[...]
    sel2 = sel2_ref[...]
    dot = lambda a, b: jnp.dot(a, b, preferred_element_type=jnp.float32)
    for m in range(TC_BLK):
        p1 = dot(stg1_ref[m], oh1_ref[m])
        ang1 = dot(sel, dot(p1, bc) * msk)
        p2 = dot(stg2_ref[m], oh2_ref[m])
        ang2 = dot(sel2, dot(p2, b2) * msk2)
        out_ref[m] = ang1 + ang2


@jax.jit
def _aev(labels, coords, pj, pk):
    mesh = plsc.VectorSubcoreMesh(core_axis_name="c", subcore_axis_name="s")
    sc_fn = pl.kernel(
        _sc_body,
        out_type=(
            jax.ShapeDtypeStruct((B, N * RFEAT), jnp.float32),
            jax.ShapeDtypeStruct((B, NROW, 128), jnp.float32),
            jax.ShapeDtypeStruct((B, NROW // 2, 128), jnp.float32),
            jax.ShapeDtypeStruct((B, NTRI_PAD), jnp.int32),
        ),
        mesh=mesh,
        scratch_types=[
            pltpu.VMEM((N,), jnp.int32),
            pltpu.VMEM((N, 3), jnp.float32),
            pltpu.VMEM((NTRI_PAD,), jnp.int32),
            pltpu.VMEM((NTRI_PAD,), jnp.int32),
            pltpu.VMEM((NP2,), jnp.float32),
            pltpu.VMEM((NP2,), jnp.float32),
            pltpu.VMEM((NP2,), jnp.float32),
            pltpu.VMEM((NTRI_PAD,), jnp.int32),
            pltpu.VMEM((N * RFEAT,), jnp.float32),
            pltpu.VMEM((NROW, 128), jnp.float32),
            pltpu.VMEM((NROW // 2, 128), jnp.float32),
        ],
        compiler_params=pltpu.CompilerParams(needs_layout_passes=False),
    )
    rad, stg1, stg2, pb = sc_fn(labels, coords, pj, pk)

    oh = (pb[:, :, None] == jnp.arange(NPB, dtype=jnp.int32)
          ).astype(jnp.float32)
    oh1 = oh[:, :128, :]
    hi = oh[:, 128:, :]
    zz = jnp.zeros((B, 64, NPB), jnp.float32)
    oh2 = jnp.concatenate([jnp.concatenate([hi, zz], 2),
                           jnp.concatenate([zz, hi], 2)], 1)
    full = lambda g: (0, 0)
    ang = pl.pallas_call(
        _tc_seg_body,
        grid=(B // TC_BLK,),
        in_specs=[
            pl.BlockSpec((TC_BLK, 128, NPB), lambda g: (g, 0, 0)),
            pl.BlockSpec((TC_BLK, 128, 2 * NPB), lambda g: (g, 0, 0)),
            pl.BlockSpec((TC_BLK, NROW, 128), lambda g: (g, 0, 0)),
            pl.BlockSpec((TC_BLK, NROW // 2, 128), lambda g: (g, 0, 0)),
            pl.BlockSpec((NPB, NPB * NSUB), full),
            pl.BlockSpec((NROW, NPB * NSUB), full),
            pl.BlockSpec((N, NROW), full),
            pl.BlockSpec((2 * NPB, NPB * NSUB), full),
            pl.BlockSpec((NROW // 2, NPB * NSUB), full),
            pl.BlockSpec((N, NROW // 2), full),
        ],
        out_specs=pl.BlockSpec((TC_BLK, N, NPB * NSUB), lambda g: (g, 0, 0)),
        out_shape=jax.ShapeDtypeStruct((B, N, NPB * NSUB), jnp.float32),
    )(oh1, oh2, stg1, stg2,
      jnp.asarray(SEL_BC), jnp.asarray(SEL_MSK), jnp.asarray(SEL_SEL),
      jnp.asarray(SEL_B2), jnp.asarray(SEL_MSK2), jnp.asarray(SEL_SEL2))
    return jnp.concatenate([rad.reshape(B, N, RFEAT), ang], axis=-1)


def kernel(labels_tensor, coords_tensor):
    pj = jnp.asarray(PAIR_J)
    pk = jnp.asarray(PAIR_K)
    aev = _aev(labels_tensor, coords_tensor, pj, pk)
    return (labels_tensor, aev)
